# baseline (speedup 1.0000x reference)
import jax
import jax.numpy as jnp
from jax import lax
from jax.experimental import pallas as pl
from jax.experimental.pallas import tpu as pltpu
from jax.experimental.pallas import tpu_sc as plsc

_L = 200
_B = 4096
_D = 64
_V = 1000000
_BC = 256
_CHW = _B // _BC
_NU = _L * _CHW
_NC = 2
_NS = 16
_NW = _NC * _NS
_UW = _NU // _NW
_NG = _BC // 128


_CC = 632
_CSTEP = 624
_CPW = 31250
_GR = 51


def _repack_body(et_hbm, t2_hbm, tbuf, obuf, isem, osem):
    c = lax.axis_index("c")
    s = lax.axis_index("s")
    wid = s * _NC + c
    col_base = wid * _CPW
    iota16 = lax.iota(jnp.int32, 16)
    zeros16 = jnp.zeros((16,), jnp.int32)

    def chunk_c0(g):
        c0 = ((col_base + g * _CSTEP) >> 3) << 3
        return pl.multiple_of(jnp.minimum(c0, _V - _CC), 8)

    def fire_in(g, s2):
        c0 = chunk_c0(g)
        pltpu.async_copy(et_hbm.at[:, pl.ds(c0, _CC)],
                         tbuf.at[s2, :, pl.ds(0, _CC)], isem.at[s2])

    def wait_in(g, s2):
        c0 = chunk_c0(g)
        pltpu.make_async_copy(et_hbm.at[:, pl.ds(c0, _CC)],
                              tbuf.at[s2, :, pl.ds(0, _CC)],
                              isem.at[s2]).wait()

    fire_in(0, 0)
    fire_in(1, 1)

    def body(g, carry):
        s2 = lax.rem(g, 2)
        wait_in(g, s2)

        @pl.when(g >= 1)
        def _():
            pltpu.make_async_copy(
                obuf.at[pl.ds(0, _CC), pl.ds(0, _D)],
                t2_hbm.at[pl.ds(chunk_c0(g - 1), _CC), 0], osem).wait()

        @plsc.parallel_loop(0, _D * 10, 1, unroll=4)
        def _(q):
            d = q // 10
            m = lax.rem(q, 10) * 4
            dv = zeros16 + d
            for r in range(4):
                lanev = iota16 + (m + r) * 16
                vals = tbuf[s2, d, pl.ds((m + r) * 16, 16)]
                plsc.store_scatter(obuf, [lanev, dv], vals,
                                   mask=lanev < _CC)

        pltpu.async_copy(obuf.at[pl.ds(0, _CC), pl.ds(0, _D)],
                         t2_hbm.at[pl.ds(chunk_c0(g), _CC), 0], osem)

        @pl.when(g + 2 < _GR)
        def _():
            fire_in(g + 2, s2)

        return carry

    lax.fori_loop(0, _GR, body, 0)
    pltpu.make_async_copy(
        obuf.at[pl.ds(0, _CC), pl.ds(0, _D)],
        t2_hbm.at[pl.ds(chunk_c0(_GR - 1), _CC), 0], osem).wait()


def _body(xt_hbm, tbl_hbm, out_hbm, ibuf, gbuf, pbuf, isem, gsem, osem):
    c = lax.axis_index("c")
    s = lax.axis_index("s")
    wid = s * _NC + c
    u_base = wid * _UW

    iota16 = lax.iota(jnp.int32, 16)

    def unit_lb(u):
        ug = u_base + u
        return ug // _CHW, lax.rem(ug, _CHW)

    def fire_idx(u, s4):
        l, bc = unit_lb(u)
        pltpu.async_copy(xt_hbm.at[l, pl.ds(bc * _NG, _NG)], ibuf.at[s4],
                         isem.at[s4])

    def wait_idx(u, s4):
        l, bc = unit_lb(u)
        pltpu.make_async_copy(xt_hbm.at[l, pl.ds(bc * _NG, _NG)],
                              ibuf.at[s4], isem.at[s4]).wait()

    def fire_gather(s4, s2):
        for k in range(_NG):
            pltpu.async_copy(tbl_hbm.at[ibuf.at[s4, k]],
                             gbuf.at[s2, pl.ds(k * 128, 128)], gsem.at[s2])

    def drain_gather(s4, s2):
        for k in range(_NG):
            pltpu.make_async_copy(tbl_hbm.at[ibuf.at[s4, k]],
                                  gbuf.at[s2, pl.ds(k * 128, 128)],
                                  gsem.at[s2]).wait()

    def fire_out(u, s2):
        l, bc = unit_lb(u)
        pltpu.async_copy(pbuf.at[s2, :, :, :, pl.ds(0, 128)],
                         out_hbm.at[l, :, pl.ds(bc * 2, 2)],
                         osem.at[s2])

    def drain_out(u, s2):
        l, bc = unit_lb(u)
        pltpu.make_async_copy(pbuf.at[s2, :, :, :, pl.ds(0, 128)],
                              out_hbm.at[l, :, pl.ds(bc * 2, 2)],
                              osem.at[s2]).wait()

    zeros16 = jnp.zeros((16,), jnp.int32)
    dgv = [(iota16 + cc * 16) >> 3 for cc in range(_D // 16)]
    dsv = [(iota16 + cc * 16) & 7 for cc in range(_D // 16)]

    def transpose_scale(s2):
        svec = zeros16 + s2

        @plsc.parallel_loop(0, _BC, 1, unroll=8)
        def _(b):
            btv = zeros16 + (b >> 7)
            lnv = zeros16 + (b & 127)
            for cc in range(_D // 16):
                vals = gbuf[s2, b, pl.ds(cc * 16, 16)] * 8.0
                plsc.store_scatter(pbuf, [svec, dgv[cc], btv, dsv[cc],
                                          lnv], vals)

    fire_idx(0, 0)
    fire_idx(1, 1)
    fire_idx(2, 2)
    wait_idx(0, 0)
    fire_gather(0, 0)

    def body(u, carry):
        s2 = lax.rem(u, 2)
        s4 = lax.rem(u, 4)
        drain_gather(s4, s2)

        @pl.when(u + 1 < _UW)
        def _():
            wait_idx(u + 1, lax.rem(u + 1, 4))
            fire_gather(lax.rem(u + 1, 4), lax.rem(u + 1, 2))

        @pl.when(u >= 2)
        def _():
            drain_out(u - 2, s2)

        transpose_scale(s2)
        fire_out(u, s2)

        @pl.when(u + 3 < _UW)
        def _():
            fire_idx(u + 3, lax.rem(u + 3, 4))

        return carry

    lax.fori_loop(0, _UW, body, 0)
    drain_out(_UW - 2, lax.rem(_UW - 2, 2))
    drain_out(_UW - 1, lax.rem(_UW - 1, 2))


def kernel(x, embedding):
    xt = (jnp.transpose(x).astype(jnp.int32) * 2).reshape(_L, _B // 128, 128)
    mesh = plsc.VectorSubcoreMesh(core_axis_name="c", subcore_axis_name="s")
    t2 = pl.kernel(
        _repack_body,
        out_type=jax.ShapeDtypeStruct((_V, 2, _D), jnp.float32),
        mesh=mesh,
        scratch_types=[
            pltpu.VMEM((2, _D, 648), jnp.float32),
            pltpu.VMEM((_CC + 8, 65), jnp.float32),
            pltpu.SemaphoreType.DMA((2,)),
            pltpu.SemaphoreType.DMA,
        ],
        compiler_params=pltpu.CompilerParams(use_tc_tiling_on_sc=False,
                                             needs_layout_passes=False),
    )(jnp.transpose(embedding))
    tbl = t2.reshape(2 * _V, _D)
    p = pl.kernel(
        _body,
        out_type=jax.ShapeDtypeStruct((_L, _D // 8, _B // 128, 8, 128),
                                      jnp.float32),
        mesh=mesh,
        scratch_types=[
            pltpu.VMEM((4, _NG, 128), jnp.int32),
            pltpu.VMEM((2, _BC, _D), jnp.float32),
            pltpu.VMEM((2, 8, 2, 8, 129), jnp.float32),
            pltpu.SemaphoreType.DMA((4,)),
            pltpu.SemaphoreType.DMA((2,)),
            pltpu.SemaphoreType.DMA((2,)),
        ],
        compiler_params=pltpu.CompilerParams(use_tc_tiling_on_sc=False,
                                             needs_layout_passes=False),
    )(xt, tbl)
    return jnp.transpose(p, (2, 4, 0, 1, 3)).reshape(_B, _L, _D)


# speedup vs baseline: 8.0002x; 8.0002x over previous
import jax
import jax.numpy as jnp
from jax import lax
from jax.experimental import pallas as pl
from jax.experimental.pallas import tpu as pltpu
from jax.experimental.pallas import tpu_sc as plsc

_L = 200
_B = 4096
_D = 64
_V = 1000000
_BC = 256
_CHW = _B // _BC
_NU = _L * _CHW
_NC = 2
_NS = 16
_NW = _NC * _NS
_UW = _NU // _NW
_NG = _BC // 128


def _body(xt_hbm, tbl_hbm, out_hbm, ibuf, gbuf, pbuf, isem, gsem, osem):
    c = lax.axis_index("c")
    s = lax.axis_index("s")
    wid = s * _NC + c
    u_base = wid * _UW

    iota16 = lax.iota(jnp.int32, 16)

    def unit_lb(u):
        ug = u_base + u
        return ug // _CHW, lax.rem(ug, _CHW)

    def fire_idx(u, s4):
        l, bc = unit_lb(u)
        pltpu.async_copy(xt_hbm.at[l, pl.ds(bc * _NG, _NG)], ibuf.at[s4],
                         isem.at[s4])

    def wait_idx(u, s4):
        l, bc = unit_lb(u)
        pltpu.make_async_copy(xt_hbm.at[l, pl.ds(bc * _NG, _NG)],
                              ibuf.at[s4], isem.at[s4]).wait()

    def fire_gather(s4, s2):
        for k in range(_NG):
            pltpu.async_copy(tbl_hbm.at[ibuf.at[s4, k]],
                             gbuf.at[s2, pl.ds(k * 128, 128)], gsem.at[s2])

    def drain_gather(s4, s2):
        for k in range(_NG):
            pltpu.make_async_copy(tbl_hbm.at[ibuf.at[s4, k]],
                                  gbuf.at[s2, pl.ds(k * 128, 128)],
                                  gsem.at[s2]).wait()

    def fire_out(u, s2):
        l, bc = unit_lb(u)
        pltpu.async_copy(pbuf.at[s2, :, :, :, pl.ds(0, 128)],
                         out_hbm.at[l, :, pl.ds(bc * 2, 2)],
                         osem.at[s2])

    def drain_out(u, s2):
        l, bc = unit_lb(u)
        pltpu.make_async_copy(pbuf.at[s2, :, :, :, pl.ds(0, 128)],
                              out_hbm.at[l, :, pl.ds(bc * 2, 2)],
                              osem.at[s2]).wait()

    zeros16 = jnp.zeros((16,), jnp.int32)
    dgv = [(iota16 + cc * 16) >> 3 for cc in range(_D // 16)]
    dsv = [(iota16 + cc * 16) & 7 for cc in range(_D // 16)]

    def transpose_scale(s2):
        svec = zeros16 + s2

        @plsc.parallel_loop(0, _BC, 1, unroll=8)
        def _(b):
            btv = zeros16 + (b >> 7)
            lnv = zeros16 + (b & 127)
            for cc in range(_D // 16):
                vals = gbuf[s2, b, pl.ds(cc * 16, 16)] * 8.0
                plsc.store_scatter(pbuf, [svec, dgv[cc], btv, dsv[cc],
                                          lnv], vals)

    fire_idx(0, 0)
    fire_idx(1, 1)
    fire_idx(2, 2)
    wait_idx(0, 0)
    fire_gather(0, 0)

    def body(u, carry):
        s2 = lax.rem(u, 2)
        s4 = lax.rem(u, 4)
        drain_gather(s4, s2)

        @pl.when(u + 1 < _UW)
        def _():
            wait_idx(u + 1, lax.rem(u + 1, 4))
            fire_gather(lax.rem(u + 1, 4), lax.rem(u + 1, 2))

        @pl.when(u >= 2)
        def _():
            drain_out(u - 2, s2)

        transpose_scale(s2)
        fire_out(u, s2)

        @pl.when(u + 3 < _UW)
        def _():
            fire_idx(u + 3, lax.rem(u + 3, 4))

        return carry

    lax.fori_loop(0, _UW, body, 0)
    drain_out(_UW - 2, lax.rem(_UW - 2, 2))
    drain_out(_UW - 1, lax.rem(_UW - 1, 2))


def kernel(x, embedding):
    xt = (jnp.transpose(x).astype(jnp.int32) * 2).reshape(_L, _B // 128, 128)
    tbl = jnp.concatenate(
        [embedding, jnp.zeros((_V, _D), jnp.float32)], axis=1
    ).reshape(2 * _V, _D)
    mesh = plsc.VectorSubcoreMesh(core_axis_name="c", subcore_axis_name="s")
    p = pl.kernel(
        _body,
        out_type=jax.ShapeDtypeStruct((_L, _D // 8, _B // 128, 8, 128),
                                      jnp.float32),
        mesh=mesh,
        scratch_types=[
            pltpu.VMEM((4, _NG, 128), jnp.int32),
            pltpu.VMEM((2, _BC, _D), jnp.float32),
            pltpu.VMEM((2, 8, 2, 8, 129), jnp.float32),
            pltpu.SemaphoreType.DMA((4,)),
            pltpu.SemaphoreType.DMA((2,)),
            pltpu.SemaphoreType.DMA((2,)),
        ],
        compiler_params=pltpu.CompilerParams(use_tc_tiling_on_sc=False,
                                             needs_layout_passes=False),
    )(xt, tbl)
    return jnp.transpose(p, (2, 4, 0, 1, 3)).reshape(_B, _L, _D)
